# Initial kernel scaffold; baseline (speedup 1.0000x reference)
import jax
import jax.numpy as jnp
from jax.experimental import pallas as pl


def kernel(idx, weight):
    raise NotImplementedError("write your pallas kernel here")


# speedup vs baseline: 4.0765x; 4.0765x over previous
import functools

import jax
import jax.numpy as jnp
from jax import lax
from jax.experimental import pallas as pl
from jax.experimental.pallas import tpu as pltpu
from jax.experimental.pallas import tpu_sc as plsc

NC, NS = 2, 16
NW = NC * NS
CB = 128


@functools.partial(jax.jit, static_argnums=(2, 3))
def _gather(idx_w, table, nch, d):
    rpw = nch * CB
    mesh = plsc.VectorSubcoreMesh(core_axis_name="c", subcore_axis_name="s")

    @functools.partial(
        pl.kernel,
        out_type=jax.ShapeDtypeStruct((NW * rpw, d), jnp.float32),
        mesh=mesh,
        scratch_types=[
            pltpu.VMEM((nch, CB), jnp.int32),
            pltpu.VMEM((CB, d), jnp.float32),
            pltpu.SemaphoreType.DMA,
        ],
        compiler_params=pltpu.CompilerParams(use_tc_tiling_on_sc=False),
    )
    def k(idx_hbm, table_hbm, out_hbm, idx_v, rows_v, sem):
        wid = lax.axis_index("s") * NC + lax.axis_index("c")
        pltpu.sync_copy(idx_hbm.at[wid], idx_v)
        base = wid * rpw

        def chunk(j, carry):
            pltpu.async_copy(table_hbm.at[idx_v.at[j]], rows_v, sem).wait()
            pltpu.sync_copy(rows_v, out_hbm.at[pl.ds(base + j * CB, CB)])
            return carry

        lax.fori_loop(0, nch, chunk, 0)

    return k(idx_w, table)


def kernel(idx, weight):
    b = idx.size
    d = weight.shape[-1]
    nch = b // (NW * CB)
    idx_w = idx.reshape(NW, nch, CB).astype(jnp.int32)
    out = _gather(idx_w, weight, nch, d)
    return out.reshape(idx.shape + (d,))


# speedup vs baseline: 4.6769x; 1.1473x over previous
import functools

import jax
import jax.numpy as jnp
from jax import lax
from jax.experimental import pallas as pl
from jax.experimental.pallas import tpu as pltpu
from jax.experimental.pallas import tpu_sc as plsc

NC, NS = 2, 16
NW = NC * NS
CB = 128
NBUF = 8
LAG = 4


@functools.partial(jax.jit, static_argnums=(2, 3))
def _gather(idx_w, table, nch, d):
    rpw = nch * CB
    mesh = plsc.VectorSubcoreMesh(core_axis_name="c", subcore_axis_name="s")

    @functools.partial(
        pl.kernel,
        out_type=jax.ShapeDtypeStruct((NW * rpw, d), jnp.float32),
        mesh=mesh,
        scratch_types=[
            pltpu.VMEM((nch, CB), jnp.int32),
            pltpu.VMEM((NBUF, CB, d), jnp.float32),
            pltpu.SemaphoreType.DMA((NBUF,)),
            pltpu.SemaphoreType.DMA((NBUF,)),
        ],
        compiler_params=pltpu.CompilerParams(use_tc_tiling_on_sc=False),
    )
    def k(idx_hbm, table_hbm, out_hbm, idx_v, rows_v, gsem, osem):
        wid = lax.axis_index("s") * NC + lax.axis_index("c")
        pltpu.sync_copy(idx_hbm.at[wid], idx_v)
        base = wid * rpw

        def start_gather(j, slot):
            pltpu.async_copy(
                table_hbm.at[idx_v.at[j]], rows_v.at[slot], gsem.at[slot])

        def drain_chunk(jd, slot):
            pltpu.make_async_copy(
                table_hbm.at[idx_v.at[jd]], rows_v.at[slot], gsem.at[slot]
            ).wait()
            pltpu.async_copy(
                rows_v.at[slot], out_hbm.at[pl.ds(base + jd * CB, CB)],
                osem.at[slot])

        def wait_out(jd, slot):
            pltpu.make_async_copy(
                rows_v.at[slot], out_hbm.at[pl.ds(base + jd * CB, CB)],
                osem.at[slot]
            ).wait()

        for j in range(NBUF):
            start_gather(j, j)
            if j >= LAG:
                drain_chunk(j - LAG, j - LAG)

        def body(j, carry):
            slot = jnp.bitwise_and(j, NBUF - 1)
            jd = j - NBUF
            wait_out(jd, slot)
            start_gather(j, slot)
            jw = j - LAG
            drain_chunk(jw, jnp.bitwise_and(jw, NBUF - 1))
            return carry

        lax.fori_loop(NBUF, nch, body, 0)

        for jd in range(nch - LAG, nch):
            drain_chunk(jd, jd % NBUF)
        for jd in range(nch - NBUF, nch):
            wait_out(jd, jd % NBUF)

    return k(idx_w, table)


def kernel(idx, weight):
    b = idx.size
    d = weight.shape[-1]
    nch = b // (NW * CB)
    idx_w = idx.reshape(NW, nch, CB).astype(jnp.int32)
    out = _gather(idx_w, weight, nch, d)
    return out.reshape(idx.shape + (d,))
